# baseline (speedup 1.0000x reference)
import jax
import jax.numpy as jnp
from jax.experimental import pallas as pl
from jax.experimental.pallas import tpu as pltpu

_B = 4096
_F = 136
_K = _F // 2
_G = 250
_BS = 16


def _body(xa_ref, xb_ref, out_ref):
    xa = xa_ref[...]
    xb = xb_ref[...]
    s = jnp.sum(xa * xa + xb * xb, axis=1, keepdims=True)
    norm = jnp.maximum(jnp.sqrt(s), 1e-12)
    fa = (jnp.round(xa / norm * 250.0) + 125.0).astype(jnp.int32)
    fb = (jnp.round(xb / norm * 250.0) + 125.0).astype(jnp.int32)
    fbt = fb.T
    rows = jax.lax.broadcasted_iota(jnp.int32, (_G, _K), 0)
    cols = jax.lax.broadcasted_iota(jnp.int32, (_K, _G), 1)
    for i in range(_BS):
        at = (rows == fa[i : i + 1, :]).astype(jnp.bfloat16)
        bm = (cols == fbt[:, i : i + 1]).astype(jnp.bfloat16)
        cnt = jax.lax.dot_general(
            at, bm, (((1,), (0,)), ((), ())),
            preferred_element_type=jnp.float32,
        )
        out_ref[i] = jnp.minimum(cnt, 1.0)


@jax.jit
def kernel(x):
    xa = x[:, 0::2]
    xb = x[:, 1::2]
    grid = (_B // _BS,)
    return pl.pallas_call(
        _body,
        out_shape=jax.ShapeDtypeStruct((_B, _G, _G), jnp.float32),
        grid=grid,
        in_specs=[
            pl.BlockSpec((_BS, _K), lambda g: (g, 0)),
            pl.BlockSpec((_BS, _K), lambda g: (g, 0)),
        ],
        out_specs=pl.BlockSpec((_BS, _G, _G), lambda g: (g, 0, 0)),
        compiler_params=pltpu.CompilerParams(
            dimension_semantics=("arbitrary",),
        ),
    )(xa, xb)


# speedup vs baseline: 1.0537x; 1.0537x over previous
import jax
import jax.numpy as jnp
from jax.experimental import pallas as pl
from jax.experimental.pallas import tpu as pltpu

_B = 4096
_F = 136
_K = _F // 2
_G = 250
_BS = 32


def _body(xa_ref, xb_ref, out_ref):
    xa = xa_ref[...]
    xb = xb_ref[...]
    s = jnp.sum(xa * xa + xb * xb, axis=1, keepdims=True)
    norm = jnp.maximum(jnp.sqrt(s), 1e-12)
    fa = (jnp.round(xa / norm * 250.0) + 125.0).astype(jnp.int32)
    fb = (jnp.round(xb / norm * 250.0) + 125.0).astype(jnp.int32)
    fbt = fb.T
    rows = jax.lax.broadcasted_iota(jnp.int32, (_G, _K), 0)
    cols = jax.lax.broadcasted_iota(jnp.int32, (_K, _G), 1)
    for i in range(_BS):
        at = (rows == fa[i : i + 1, :]).astype(jnp.bfloat16)
        bm = (cols == fbt[:, i : i + 1]).astype(jnp.bfloat16)
        cnt = jax.lax.dot_general(
            at, bm, (((1,), (0,)), ((), ())),
            preferred_element_type=jnp.float32,
        )
        out_ref[i] = jnp.minimum(cnt, 1.0)


@jax.jit
def kernel(x):
    xa = x[:, 0::2]
    xb = x[:, 1::2]
    grid = (_B // _BS,)
    return pl.pallas_call(
        _body,
        out_shape=jax.ShapeDtypeStruct((_B, _G, _G), jnp.float32),
        grid=grid,
        in_specs=[
            pl.BlockSpec((_BS, _K), lambda g: (g, 0)),
            pl.BlockSpec((_BS, _K), lambda g: (g, 0)),
        ],
        out_specs=pl.BlockSpec((_BS, _G, _G), lambda g: (g, 0, 0)),
        compiler_params=pltpu.CompilerParams(
            dimension_semantics=("arbitrary",),
        ),
    )(xa, xb)
